# baseline (speedup 1.0000x reference)
import functools

import jax
import jax.numpy as jnp
from jax import lax
from jax.experimental import pallas as pl
from jax.experimental.pallas import tpu as pltpu
from jax.experimental.pallas import tpu_sc as plsc

D = 32
NC = 2
NS = 16
NW = NC * NS
CH = 128
BLK = 512
NCH = BLK // CH
TC = 1024


def _table_transpose(table_t, tail_rm):
    V = table_t.shape[1]
    n_chunks = V // TC
    v_main = n_chunks * TC
    tail = V - v_main
    mesh = plsc.VectorSubcoreMesh(core_axis_name="c", subcore_axis_name="s")

    @functools.partial(
        pl.kernel,
        mesh=mesh,
        out_type=jax.ShapeDtypeStruct((V // 4, 4 * D), jnp.float32),
        scratch_types=[
            pltpu.VMEM((2, D, TC), jnp.float32),
            pltpu.VMEM((1, TC // 4, 4 * D), jnp.float32),
            pltpu.SemaphoreType.DMA,
        ],
        compiler_params=pltpu.CompilerParams(
            use_tc_tiling_on_sc=True, needs_layout_passes=False),
    )
    def k(tab_hbm, tail_hbm, out_hbm, slab_v, rows_v, isem):
        wid = lax.axis_index("s") * NC + lax.axis_index("c")
        lanes = lax.iota(jnp.int32, 16)

        def i_copy(chunk, p):
            return pltpu.make_async_copy(
                tab_hbm.at[:, pl.ds(chunk * TC, TC)], slab_v.at[p], isem)

        def transpose(p, nrows):
            @plsc.parallel_loop(0, nrows, unroll=2)
            def tr_body(r):
                for q in range(4 * D // 16):
                    dvec = (q % 2) * 16 + lanes
                    cvec = jnp.zeros((16,), jnp.int32) + (4 * r + q // 2)
                    rows_v[0, r, pl.ds(q * 16, 16)] = plsc.load_gather(
                        slab_v.at[p], [dvec, cvec])

        n_mine = (n_chunks - 1 - wid) // NW + 1
        i_copy(wid, 0).start()

        def body(g, carry):
            chunk = wid + g * NW
            p = lax.rem(g, 2)
            i_copy(chunk, p).wait()

            @pl.when(g < n_mine - 1)
            def _():
                i_copy(chunk + NW, 1 - p).start()

            transpose(p, TC // 4)
            pltpu.sync_copy(
                rows_v.at[0],
                out_hbm.at[pl.ds(chunk * (TC // 4), TC // 4)])
            return carry

        lax.fori_loop(0, n_mine, body, 0)

        if tail:
            @pl.when(wid == 0)
            def _():
                pltpu.sync_copy(tail_hbm, rows_v.at[0, pl.ds(0, tail // 4)])
                pltpu.sync_copy(rows_v.at[0, pl.ds(0, tail // 4)],
                                out_hbm.at[pl.ds(v_main // 4, tail // 4)])

    return k(table_t, tail_rm)


def _go2vec_sc(idx, table, hist, batch):
    mesh = plsc.VectorSubcoreMesh(core_axis_name="c", subcore_axis_name="s")
    assert hist % 2 == 0 and batch == NW * BLK

    @functools.partial(
        pl.kernel,
        mesh=mesh,
        out_type=jax.ShapeDtypeStruct((hist, D, batch), jnp.float32),
        scratch_types=[
            pltpu.VMEM((hist, BLK), jnp.int32),
            pltpu.VMEM((2, BLK, D), jnp.float32),
            pltpu.VMEM((2, D, BLK), jnp.float32),
            pltpu.SemaphoreType.DMA,
            pltpu.SemaphoreType.DMA,
        ],
        compiler_params=pltpu.CompilerParams(
            use_tc_tiling_on_sc=False, needs_layout_passes=False),
    )
    def k(idx_hbm, table_hbm, out_hbm, idx_v, rows_v, trans_v, gsem, osem):
        wid = lax.axis_index("s") * NC + lax.axis_index("c")
        base = wid * BLK

        pltpu.sync_copy(idx_hbm.at[:, pl.ds(wid * BLK, BLK)], idx_v)

        def g_copy(h, p):
            return pltpu.make_async_copy(
                table_hbm.at[idx_v.at[h]], rows_v.at[p], gsem)

        def o_copy(h, p):
            return pltpu.make_async_copy(
                trans_v.at[p], out_hbm.at[h, :, pl.ds(base, BLK)], osem)

        def transpose(p):
            rows = rows_v.at[p]
            lanes = lax.iota(jnp.int32, 16)

            @plsc.parallel_loop(0, BLK // 16, unroll=2)
            def tr_body(g):
                bvec = g * 16 + lanes
                for d in range(D):
                    dvec = jnp.full((16,), d, jnp.int32)
                    trans_v[p, d, pl.ds(g * 16, 16)] = plsc.load_gather(
                        rows, [bvec, dvec])

        def step(h, p, hh):
            g_copy(h, p).wait()

            if p == 0:
                g_copy(h + 1, 1 - p).start()
            else:
                @pl.when(hh < hist // 2 - 1)
                def _():
                    g_copy(h + 1, 1 - p).start()

            @pl.when(hh >= 1)
            def _():
                o_copy(h - 2, p).wait()

            transpose(p)
            o_copy(h, p).start()

        g_copy(0, 0).start()

        def pair(hh, carry):
            step(2 * hh, 0, hh)
            step(2 * hh + 1, 1, hh)
            return carry

        lax.fori_loop(0, hist // 2, pair, 0)

        o_copy(hist - 2, 0).wait()
        o_copy(hist - 1, 1).wait()

    return k(idx, table)


@functools.partial(jax.jit, static_argnames=("hist", "batch", "vocab"))
def _impl(go, emb_weights, *, hist, batch, vocab):
    idx = go.T.astype(jnp.int32)
    v_main = (vocab // TC) * TC
    tail_rm = emb_weights[v_main:].reshape((vocab - v_main) // 4, 4 * D)
    t128 = _table_transpose(emb_weights.T, tail_rm)
    tbl_rm = t128.reshape(vocab, D)
    out = _go2vec_sc(idx, tbl_rm, hist, batch)
    return out.transpose(2, 0, 1)


def kernel(go, emb_weights):
    batch, hist = go.shape
    return _impl(go, emb_weights, hist=hist, batch=batch,
                 vocab=emb_weights.shape[0])


# speedup vs baseline: 1.1735x; 1.1735x over previous
import functools

import jax
import jax.numpy as jnp
from jax import lax
from jax.experimental import pallas as pl
from jax.experimental.pallas import tpu as pltpu
from jax.experimental.pallas import tpu_sc as plsc

D = 32
NC = 2
NS = 16
NW = NC * NS
CH = 128
BLK = 512
TC = 512


def _table_transpose(table_t, tail_rm):
    V = table_t.shape[1]
    n_chunks = V // TC
    v_main = n_chunks * TC
    tail = V - v_main
    mesh = plsc.VectorSubcoreMesh(core_axis_name="c", subcore_axis_name="s")

    @functools.partial(
        pl.kernel,
        mesh=mesh,
        out_type=jax.ShapeDtypeStruct((V // 4, 4 * D), jnp.float32),
        scratch_types=[
            pltpu.VMEM((3, D, TC), jnp.float32),
            pltpu.VMEM((3, TC // 4, 4 * D), jnp.float32),
            pltpu.SemaphoreType.DMA,
            pltpu.SemaphoreType.DMA,
        ],
        compiler_params=pltpu.CompilerParams(
            use_tc_tiling_on_sc=True, needs_layout_passes=False),
    )
    def k(tab_hbm, tail_hbm, out_hbm, slab_v, rows_v, isem, osem):
        wid = lax.axis_index("s") * NC + lax.axis_index("c")
        lanes = lax.iota(jnp.int32, 16)

        def i_copy(chunk, p):
            return pltpu.make_async_copy(
                tab_hbm.at[:, pl.ds(chunk * TC, TC)], slab_v.at[p], isem)

        def o_copy(chunk, p):
            return pltpu.make_async_copy(
                rows_v.at[p], out_hbm.at[pl.ds(chunk * (TC // 4), TC // 4)],
                osem)

        def transpose(p, nrows):
            @plsc.parallel_loop(0, nrows, unroll=2)
            def tr_body(r):
                for q in range(4 * D // 16):
                    dvec = (q % 2) * 16 + lanes
                    cvec = jnp.zeros((16,), jnp.int32) + (4 * r + q // 2)
                    rows_v[p, r, pl.ds(q * 16, 16)] = plsc.load_gather(
                        slab_v.at[p], [dvec, cvec])

        n_mine = (n_chunks - 1 - wid) // NW + 1
        i_copy(wid, 0).start()

        @pl.when(n_mine >= 2)
        def _():
            i_copy(wid + NW, 1).start()

        def body(g, carry):
            chunk = wid + g * NW
            p = lax.rem(g, 3)
            i_copy(chunk, p).wait()

            @pl.when(g < n_mine - 2)
            def _():
                i_copy(chunk + 2 * NW, lax.rem(g + 2, 3)).start()

            @pl.when(g >= 3)
            def _():
                o_copy(0, p).wait()

            transpose(p, TC // 4)
            o_copy(chunk, p).start()
            return carry

        lax.fori_loop(0, n_mine, body, 0)

        o_copy(0, 0).wait()
        o_copy(0, 1).wait()
        o_copy(0, 2).wait()

        if tail:
            @pl.when(wid == 0)
            def _():
                pltpu.sync_copy(tail_hbm, rows_v.at[0, pl.ds(0, tail // 4)])
                pltpu.sync_copy(rows_v.at[0, pl.ds(0, tail // 4)],
                                out_hbm.at[pl.ds(v_main // 4, tail // 4)])

    return k(table_t, tail_rm)


def _go2vec_sc(idx, table, hist, batch):
    mesh = plsc.VectorSubcoreMesh(core_axis_name="c", subcore_axis_name="s")
    assert hist % 2 == 0 and batch == NW * BLK
    NTC = BLK // CH

    @functools.partial(
        pl.kernel,
        mesh=mesh,
        out_type=jax.ShapeDtypeStruct((hist, D // 8, batch // CH, 8, CH),
                                      jnp.float32),
        scratch_types=[
            pltpu.VMEM((hist, BLK), jnp.int32),
            pltpu.VMEM((2, BLK, D), jnp.float32),
            pltpu.VMEM((2, D // 8, NTC, 8, CH), jnp.float32),
            pltpu.SemaphoreType.DMA,
            pltpu.SemaphoreType.DMA,
        ],
        compiler_params=pltpu.CompilerParams(
            use_tc_tiling_on_sc=False, needs_layout_passes=False),
    )
    def k(idx_hbm, table_hbm, out_hbm, idx_v, rows_v, trans_v, gsem, osem):
        wid = lax.axis_index("s") * NC + lax.axis_index("c")

        pltpu.sync_copy(idx_hbm.at[:, pl.ds(wid * BLK, BLK)], idx_v)

        def g_copy(h, p):
            return pltpu.make_async_copy(
                table_hbm.at[idx_v.at[h]], rows_v.at[p], gsem)

        def o_copy(h, p, r):
            return pltpu.make_async_copy(
                trans_v.at[p, r],
                out_hbm.at[h, r, pl.ds(wid * NTC, NTC)], osem)

        def transpose(p):
            rows = rows_v.at[p]
            lanes = lax.iota(jnp.int32, 16)

            @plsc.parallel_loop(0, BLK // 16, unroll=2)
            def tr_body(g):
                bvec = g * 16 + lanes
                ct = g // 8
                lo = lax.rem(g, 8) * 16
                for d in range(D):
                    dvec = jnp.full((16,), d, jnp.int32)
                    trans_v[p, d // 8, ct, d % 8, pl.ds(lo, 16)] = (
                        plsc.load_gather(rows, [bvec, dvec]))

        def step(h, p, hh):
            g_copy(h, p).wait()

            if p == 0:
                g_copy(h + 1, 1 - p).start()
            else:
                @pl.when(hh < hist // 2 - 1)
                def _():
                    g_copy(h + 1, 1 - p).start()

            @pl.when(hh >= 1)
            def _():
                for r in range(D // 8):
                    o_copy(h - 2, p, r).wait()

            transpose(p)
            for r in range(D // 8):
                o_copy(h, p, r).start()

        g_copy(0, 0).start()

        def pair(hh, carry):
            step(2 * hh, 0, hh)
            step(2 * hh + 1, 1, hh)
            return carry

        lax.fori_loop(0, hist // 2, pair, 0)

        for r in range(D // 8):
            o_copy(hist - 2, 0, r).wait()
            o_copy(hist - 1, 1, r).wait()

    return k(idx, table)


@functools.partial(jax.jit, static_argnames=("hist", "batch", "vocab"))
def _impl(go, emb_weights, *, hist, batch, vocab):
    idx = go.T.astype(jnp.int32)
    v_main = (vocab // TC) * TC
    tail_rm = emb_weights[v_main:].reshape((vocab - v_main) // 4, 4 * D)
    t128 = _table_transpose(emb_weights.T, tail_rm)
    tbl_rm = t128.reshape(vocab, D)
    out5 = _go2vec_sc(idx, tbl_rm, hist, batch)
    return out5.transpose(2, 4, 0, 1, 3).reshape(batch, hist, D)


def kernel(go, emb_weights):
    batch, hist = go.shape
    return _impl(go, emb_weights, hist=hist, batch=batch,
                 vocab=emb_weights.shape[0])


# speedup vs baseline: 1.1838x; 1.0087x over previous
import functools

import jax
import jax.numpy as jnp
from jax import lax
from jax.experimental import pallas as pl
from jax.experimental.pallas import tpu as pltpu
from jax.experimental.pallas import tpu_sc as plsc

D = 32
NC = 2
NS = 16
NW = NC * NS
CH = 128
BLK = 512
TC = 512


def _table_transpose(table_t, tail_rm):
    V = table_t.shape[1]
    n_chunks = V // TC
    v_main = n_chunks * TC
    tail = V - v_main
    mesh = plsc.VectorSubcoreMesh(core_axis_name="c", subcore_axis_name="s")

    @functools.partial(
        pl.kernel,
        mesh=mesh,
        out_type=jax.ShapeDtypeStruct((V // 4, 4 * D), jnp.float32),
        scratch_types=[
            pltpu.VMEM((3, D, TC), jnp.float32),
            pltpu.VMEM((3, TC // 4, 4 * D), jnp.float32),
            pltpu.SemaphoreType.DMA,
            pltpu.SemaphoreType.DMA,
        ],
        compiler_params=pltpu.CompilerParams(
            use_tc_tiling_on_sc=True, needs_layout_passes=False),
    )
    def k(tab_hbm, tail_hbm, out_hbm, slab_v, rows_v, isem, osem):
        wid = lax.axis_index("s") * NC + lax.axis_index("c")
        lanes = lax.iota(jnp.int32, 16)

        def i_copy(chunk, p):
            return pltpu.make_async_copy(
                tab_hbm.at[:, pl.ds(chunk * TC, TC)], slab_v.at[p], isem)

        def o_copy(chunk, p):
            return pltpu.make_async_copy(
                rows_v.at[p], out_hbm.at[pl.ds(chunk * (TC // 4), TC // 4)],
                osem)

        def transpose(p, nrows):
            @plsc.parallel_loop(0, nrows, unroll=2)
            def tr_body(r):
                for q in range(4 * D // 16):
                    dvec = (q % 2) * 16 + lanes
                    cvec = jnp.zeros((16,), jnp.int32) + (4 * r + q // 2)
                    rows_v[p, r, pl.ds(q * 16, 16)] = plsc.load_gather(
                        slab_v.at[p], [dvec, cvec])

        n_mine = (n_chunks - 1 - wid) // NW + 1
        i_copy(wid, 0).start()

        @pl.when(n_mine >= 2)
        def _():
            i_copy(wid + NW, 1).start()

        def body(g, carry):
            chunk = wid + g * NW
            p = lax.rem(g, 3)
            i_copy(chunk, p).wait()

            @pl.when(g < n_mine - 2)
            def _():
                i_copy(chunk + 2 * NW, lax.rem(g + 2, 3)).start()

            @pl.when(g >= 3)
            def _():
                o_copy(0, p).wait()

            transpose(p, TC // 4)
            o_copy(chunk, p).start()
            return carry

        lax.fori_loop(0, n_mine, body, 0)

        o_copy(0, 0).wait()
        o_copy(0, 1).wait()
        o_copy(0, 2).wait()

        if tail:
            @pl.when(wid == 0)
            def _():
                pltpu.sync_copy(tail_hbm, rows_v.at[0, pl.ds(0, tail // 4)])
                pltpu.sync_copy(rows_v.at[0, pl.ds(0, tail // 4)],
                                out_hbm.at[pl.ds(v_main // 4, tail // 4)])

    return k(table_t, tail_rm)


def _go2vec_sc(idx, table, hist, batch):
    mesh = plsc.VectorSubcoreMesh(core_axis_name="c", subcore_axis_name="s")
    assert hist % 2 == 0 and batch == NW * BLK
    NTC = BLK // CH

    @functools.partial(
        pl.kernel,
        mesh=mesh,
        out_type=jax.ShapeDtypeStruct((hist, D // 8, batch // CH, 8, CH),
                                      jnp.float32),
        scratch_types=[
            pltpu.VMEM((hist, BLK), jnp.int32),
            pltpu.VMEM((4, BLK, D), jnp.float32),
            pltpu.VMEM((2, D // 8, NTC, 8, CH), jnp.float32),
            pltpu.SemaphoreType.DMA,
            pltpu.SemaphoreType.DMA,
        ],
        compiler_params=pltpu.CompilerParams(
            use_tc_tiling_on_sc=False, needs_layout_passes=False),
    )
    def k(idx_hbm, table_hbm, out_hbm, idx_v, rows_v, trans_v, gsem, osem):
        wid = lax.axis_index("s") * NC + lax.axis_index("c")

        pltpu.sync_copy(idx_hbm.at[:, pl.ds(wid * BLK, BLK)], idx_v)

        def g_copy(h, p):
            return pltpu.make_async_copy(
                table_hbm.at[idx_v.at[h]], rows_v.at[p], gsem)

        def o_copy(h, p, r):
            return pltpu.make_async_copy(
                trans_v.at[p, r],
                out_hbm.at[h, r, pl.ds(wid * NTC, NTC)], osem)

        def transpose(pr, tp):
            rows = rows_v.at[pr]
            lanes = lax.iota(jnp.int32, 16)

            @plsc.parallel_loop(0, BLK // 16, unroll=2)
            def tr_body(g):
                bvec = g * 16 + lanes
                ct = g // 8
                lo = lax.rem(g, 8) * 16
                for d in range(D):
                    dvec = jnp.full((16,), d, jnp.int32)
                    trans_v[tp, d // 8, ct, d % 8, pl.ds(lo, 16)] = (
                        plsc.load_gather(rows, [bvec, dvec]))

        def step(h, tp, hh):
            pr = lax.rem(h, 4)
            g_copy(h, pr).wait()

            @pl.when(h + 3 < hist)
            def _():
                g_copy(h + 3, lax.rem(h + 3, 4)).start()

            @pl.when(hh >= 1)
            def _():
                for r in range(D // 8):
                    o_copy(h - 2, tp, r).wait()

            transpose(pr, tp)
            for r in range(D // 8):
                o_copy(h, tp, r).start()

        g_copy(0, 0).start()
        g_copy(1, 1).start()
        g_copy(2, 2).start()

        def pair(hh, carry):
            step(2 * hh, 0, hh)
            step(2 * hh + 1, 1, hh)
            return carry

        lax.fori_loop(0, hist // 2, pair, 0)

        for r in range(D // 8):
            o_copy(hist - 2, 0, r).wait()
            o_copy(hist - 1, 1, r).wait()

    return k(idx, table)


@functools.partial(jax.jit, static_argnames=("hist", "batch", "vocab"))
def _impl(go, emb_weights, *, hist, batch, vocab):
    idx = go.T.astype(jnp.int32)
    v_main = (vocab // TC) * TC
    tail_rm = emb_weights[v_main:].reshape((vocab - v_main) // 4, 4 * D)
    t128 = _table_transpose(emb_weights.T, tail_rm)
    tbl_rm = t128.reshape(vocab, D)
    out5 = _go2vec_sc(idx, tbl_rm, hist, batch)
    return out5.transpose(2, 4, 0, 1, 3).reshape(batch, hist, D)


def kernel(go, emb_weights):
    batch, hist = go.shape
    return _impl(go, emb_weights, hist=hist, batch=batch,
                 vocab=emb_weights.shape[0])
